# baseline (speedup 1.0000x reference)
import functools

import jax
import jax.numpy as jnp
from jax import lax
from jax.experimental import pallas as pl
from jax.experimental.pallas import tpu as pltpu
from jax.experimental.pallas import tpu_sc as plsc

D_MODEL = 64
SCALE = float(D_MODEL) ** 0.5

NC = 2
NS = 16
NW = NC * NS

BLK = 128


def _emb_kernel(n_total: int):
    assert n_total % (NW * BLK) == 0
    per_w = n_total // NW
    n_blk = per_w // BLK

    mesh = plsc.VectorSubcoreMesh(core_axis_name="c", subcore_axis_name="s")

    @functools.partial(
        pl.kernel,
        out_type=jax.ShapeDtypeStruct((n_total, D_MODEL), jnp.float32),
        mesh=mesh,
        scratch_types=[
            pltpu.VMEM((n_blk, BLK), jnp.int32),
            pltpu.VMEM((BLK, D_MODEL), jnp.float32),
            pltpu.SemaphoreType.DMA,
        ],
        compiler_params=pltpu.CompilerParams(use_tc_tiling_on_sc=False),
    )
    def k(x_hbm, lut_hbm, out_hbm, idx_v, rows_v, sem):
        wid = lax.axis_index("s") * NC + lax.axis_index("c")
        pltpu.sync_copy(x_hbm.at[pl.ds(wid * n_blk, n_blk)], idx_v)

        def step(j, carry):
            pltpu.async_copy(lut_hbm.at[idx_v.at[j]], rows_v, sem).wait()

            def srow(i, c2):
                for t in range(D_MODEL // 16):
                    sl = pl.ds(t * 16, 16)
                    rows_v[i, sl] = rows_v[i, sl] * SCALE
                return c2

            lax.fori_loop(0, BLK, srow, 0)
            pltpu.sync_copy(
                rows_v, out_hbm.at[pl.ds(wid * per_w + j * BLK, BLK)])
            return carry

        lax.fori_loop(0, n_blk, step, 0)

    return k


def kernel(x, lut):
    b, s = x.shape
    n_total = b * s
    x2d = x.reshape(NW * (n_total // (NW * BLK)), BLK)
    out = _emb_kernel(n_total)(x2d, lut)
    return out.reshape(b, s, D_MODEL)


# speedup vs baseline: 1.2100x; 1.2100x over previous
import functools

import jax
import jax.numpy as jnp
from jax import lax
from jax.experimental import pallas as pl
from jax.experimental.pallas import tpu as pltpu
from jax.experimental.pallas import tpu_sc as plsc

D_MODEL = 64
SCALE = float(D_MODEL) ** 0.5

NC = 2
NS = 16
NW = NC * NS

BLK = 128
NBUF = 4


def _emb_kernel(n_total: int):
    assert n_total % (NW * BLK) == 0
    per_w = n_total // NW
    n_blk = per_w // BLK
    assert n_blk % NBUF == 0 and n_blk >= 2 * NBUF

    mesh = plsc.VectorSubcoreMesh(core_axis_name="c", subcore_axis_name="s")

    @functools.partial(
        pl.kernel,
        out_type=jax.ShapeDtypeStruct((n_total, D_MODEL), jnp.float32),
        mesh=mesh,
        scratch_types=[
            pltpu.VMEM((n_blk, BLK), jnp.int32),
            pltpu.VMEM((NBUF, BLK, D_MODEL), jnp.float32),
            pltpu.VMEM((NBUF, BLK, D_MODEL), jnp.float32),
            [pltpu.SemaphoreType.DMA] * NBUF,
            [pltpu.SemaphoreType.DMA] * NBUF,
        ],
        compiler_params=pltpu.CompilerParams(use_tc_tiling_on_sc=False),
    )
    def k(x_hbm, lut_hbm, out_hbm, idx_v, gbuf, sbuf, gsems, ssems):
        wid = lax.axis_index("s") * NC + lax.axis_index("c")
        out_w = out_hbm.at[pl.ds(wid * per_w, per_w)]
        pltpu.sync_copy(x_hbm.at[pl.ds(wid * n_blk, n_blk)], idx_v)

        def start_gather(j, b):
            pltpu.async_copy(lut_hbm.at[idx_v.at[j]], gbuf.at[b], gsems[b])

        def wait_gather(j, b):
            pltpu.make_async_copy(
                lut_hbm.at[idx_v.at[j]], gbuf.at[b], gsems[b]).wait()

        def start_store(j, b):
            pltpu.async_copy(
                sbuf.at[b], out_w.at[pl.ds(j * BLK, BLK)], ssems[b])

        def wait_store(j, b):
            pltpu.make_async_copy(
                sbuf.at[b], out_w.at[pl.ds(j * BLK, BLK)], ssems[b]).wait()

        def scale(b):
            g = gbuf.at[b]
            s = sbuf.at[b]

            @plsc.parallel_loop(0, BLK, unroll=4)
            def _(i):
                for t in range(D_MODEL // 16):
                    sl = pl.ds(t * 16, 16)
                    s[i, sl] = g[i, sl] * SCALE

        for b in range(NBUF):
            start_gather(b, b)

        for b in range(NBUF):
            wait_gather(b, b)
            scale(b)
            start_store(b, b)
            start_gather(b + NBUF, b)

        def outer(jo, carry):
            j0 = NBUF + jo * NBUF
            for b in range(NBUF):
                j = j0 + b
                wait_gather(j, b)
                wait_store(j - NBUF, b)
                scale(b)
                start_store(j, b)
                start_gather(j + NBUF, b)
            return carry

        lax.fori_loop(0, (n_blk - 2 * NBUF) // NBUF, outer, 0)

        for b in range(NBUF):
            j = n_blk - NBUF + b
            wait_gather(j, b)
            wait_store(j - NBUF, b)
            scale(b)
            start_store(j, b)

        for b in range(NBUF):
            wait_store(n_blk - NBUF + b, b)

    return k


def kernel(x, lut):
    b, s = x.shape
    n_total = b * s
    x2d = x.reshape(NW * (n_total // (NW * BLK)), BLK)
    out = _emb_kernel(n_total)(x2d, lut)
    return out.reshape(b, s, D_MODEL)
